# baseline (speedup 1.0000x reference)
import jax
import jax.numpy as jnp
from jax import lax
from jax.experimental import pallas as pl
from jax.experimental.pallas import tpu as pltpu
from jax.experimental.pallas import tpu_sc as plsc
import functools

NP = 256
H = 768
NB = 32
BI = 32
U_ROWS = 512
MID0, MID = 192, 128
NC, NS = 2, 16
NW = NC * NS
RPW = MID // NW


def _sc_lookup_body(tab_hbm, u2_hbm, idx, rows, sem):
    wid = lax.axis_index("s") * NC + lax.axis_index("c")
    k0 = MID0 + wid * RPW

    lane = lax.broadcasted_iota(jnp.int32, (16,), 0)
    kv = k0 + lax.rem(lane, RPW)
    d = jnp.abs(255 - kv)
    c = jnp.minimum(d, NB)
    b = (NB + 1) + jnp.minimum(lax.shift_right_logical(d, 1), NB - 1)
    idx[...] = jnp.where(lane < RPW, c, b)

    pltpu.async_copy(tab_hbm.at[idx.at[pl.ds(0, 2 * RPW)]], rows, sem).wait()
    pltpu.async_copy(rows, u2_hbm.at[wid], sem).wait()


@functools.partial(
    pl.kernel,
    out_type=jax.ShapeDtypeStruct((NW, 2 * RPW, H), jnp.float32),
    mesh=plsc.VectorSubcoreMesh(core_axis_name="c", subcore_axis_name="s"),
    scratch_types=[
        pltpu.VMEM((16,), jnp.int32),
        pltpu.VMEM((2 * RPW, H), jnp.float32),
        pltpu.SemaphoreType.DMA,
    ],
)
def _sc_lookup(*args):
    _sc_lookup_body(*args)


def _tc_broadcast(u2_ref, out_ref, u_ref):
    pid = pl.program_id(0)

    @pl.when(pid == 0)
    def _build_u():
        u2 = u2_ref[...]
        mid = (u2[:, :RPW, :] + u2[:, RPW:, :]).reshape(MID, H)
        const = jnp.broadcast_to(mid[0:1], (U_ROWS - MID0 - MID, H))
        u_ref[pl.ds(0, MID0), :] = const[:MID0]
        u_ref[pl.ds(MID0, MID), :] = mid
        u_ref[pl.ds(MID0 + MID, U_ROWS - MID0 - MID), :] = const

    for ii in range(BI):
        s = 255 - (pid * BI + ii)
        r = (7 - ii) % 8
        q8 = pl.multiple_of(s - r, 8)
        tmp = u_ref[pl.ds(q8, NP + 8), :]
        out_ref[ii] = tmp[r:r + NP]


@jax.jit
def kernel(token_embeddings, rel_height, rel_width):
    tab = jnp.concatenate(
        [jnp.concatenate([rel_height, rel_width], axis=1), token_embeddings],
        axis=0)
    u2 = _sc_lookup(tab)
    return pl.pallas_call(
        _tc_broadcast,
        grid=(NP // BI,),
        in_specs=[pl.BlockSpec((NW, 2 * RPW, H), lambda i: (0, 0, 0))],
        out_specs=pl.BlockSpec((BI, NP, H), lambda i: (i, 0, 0)),
        out_shape=jax.ShapeDtypeStruct((NP, NP, H), jnp.float32),
        scratch_shapes=[pltpu.VMEM((U_ROWS, H), jnp.float32)],
    )(u2)


# speedup vs baseline: 1.0231x; 1.0231x over previous
import jax
import jax.numpy as jnp
from jax import lax
from jax.experimental import pallas as pl
from jax.experimental.pallas import tpu as pltpu
from jax.experimental.pallas import tpu_sc as plsc
import functools

NP = 256
H = 768
NB = 32
BI = 8
U_ROWS = 512
MID0, MID = 192, 128
NC, NS = 2, 16
NW = NC * NS
RPW = MID // NW


def _sc_lookup_body(tab_hbm, u2_hbm, idx, rows, sem):
    wid = lax.axis_index("s") * NC + lax.axis_index("c")
    k0 = MID0 + wid * RPW

    lane = lax.broadcasted_iota(jnp.int32, (16,), 0)
    kv = k0 + lax.rem(lane, RPW)
    d = jnp.abs(255 - kv)
    c = jnp.minimum(d, NB)
    b = (NB + 1) + jnp.minimum(lax.shift_right_logical(d, 1), NB - 1)
    idx[...] = jnp.where(lane < RPW, c, b)

    pltpu.async_copy(tab_hbm.at[idx.at[pl.ds(0, 2 * RPW)]], rows, sem).wait()
    pltpu.async_copy(rows, u2_hbm.at[wid], sem).wait()


@functools.partial(
    pl.kernel,
    out_type=jax.ShapeDtypeStruct((NW, 2 * RPW, H), jnp.float32),
    mesh=plsc.VectorSubcoreMesh(core_axis_name="c", subcore_axis_name="s"),
    scratch_types=[
        pltpu.VMEM((16,), jnp.int32),
        pltpu.VMEM((2 * RPW, H), jnp.float32),
        pltpu.SemaphoreType.DMA,
    ],
)
def _sc_lookup(*args):
    _sc_lookup_body(*args)


def _tc_broadcast(u2_ref, out_ref, u_ref):
    pid = pl.program_id(0)

    @pl.when(pid == 0)
    def _build_u():
        u2 = u2_ref[...]
        mid = (u2[:, :RPW, :] + u2[:, RPW:, :]).reshape(MID, H)
        const = jnp.broadcast_to(mid[0:1], (U_ROWS - MID0 - MID, H))
        u_ref[pl.ds(0, MID0), :] = const[:MID0]
        u_ref[pl.ds(MID0, MID), :] = mid
        u_ref[pl.ds(MID0 + MID, U_ROWS - MID0 - MID), :] = const

    for ii in range(BI):
        s = 255 - (pid * BI + ii)
        r = (7 - ii) % 8
        q8 = pl.multiple_of(s - r, 8)
        tmp = u_ref[pl.ds(q8, NP + 8), :]
        out_ref[ii] = tmp[r:r + NP]


@jax.jit
def kernel(token_embeddings, rel_height, rel_width):
    tab = jnp.concatenate(
        [jnp.concatenate([rel_height, rel_width], axis=1), token_embeddings],
        axis=0)
    u2 = _sc_lookup(tab)
    return pl.pallas_call(
        _tc_broadcast,
        grid=(NP // BI,),
        in_specs=[pl.BlockSpec((NW, 2 * RPW, H), lambda i: (0, 0, 0))],
        out_specs=pl.BlockSpec((BI, NP, H), lambda i: (i, 0, 0)),
        out_shape=jax.ShapeDtypeStruct((NP, NP, H), jnp.float32),
        scratch_shapes=[pltpu.VMEM((U_ROWS, H), jnp.float32)],
    )(u2)
